# baseline (device time: 44165 ns/iter reference)
import functools

import jax
import jax.numpy as jnp
from jax import lax
from jax.experimental import pallas as pl
from jax.experimental.pallas import tpu as pltpu

N_DEV = 8

PIPES = (
    ((4, 3, 1), [0, 4, 6, 2, 1, 5, 7, 3]),
    ((3, 1, 4), [0, 2, 3, 1, 4, 6, 7, 5]),
    ((1, 4, 3), [0, 1, 5, 4, 2, 3, 7, 6]),
)
SPLITS = ((0, 96), (96, 80), (176, 80))


def kernel(x, w_mat):
    m_per, k = x.shape
    _, n_per = w_mat.shape

    def body(x_ref, w_ref, out_ref, xg0_ref, xg1_ref, xg2_ref,
             xv_ref, wv_ref, send_sems, recv_sems, copy_sems):
        my = lax.axis_index("i")

        xcp = pltpu.make_async_copy(x_ref, xv_ref, copy_sems.at[0])
        wcp = pltpu.make_async_copy(w_ref, wv_ref, copy_sems.at[1])
        xcp.start()
        wcp.start()
        b0 = jnp.bitwise_and(my, 1)
        b1 = jnp.bitwise_and(jnp.right_shift(my, 1), 1)
        b2 = jnp.bitwise_and(jnp.right_shift(my, 2), 1)
        b01 = jnp.bitwise_xor(b0, b1)
        refs = [xg0_ref, xg1_ref, xg2_ref]
        ts = [b2 + 2 * b1 + 4 * b01, b1 + 2 * b01 + 4 * b2,
              b01 + 2 * b2 + 4 * b1]

        barrier = pltpu.get_barrier_semaphore()
        for msk in (1, 3, 4):
            pl.semaphore_signal(
                barrier, inc=1,
                device_id=(jnp.bitwise_xor(my, msk),),
                device_id_type=pl.DeviceIdType.MESH,
            )
        pl.semaphore_wait(barrier, 3)

        xcp.wait()
        for P in range(3):
            r0, nr = SPLITS[P]
            refs[P][ts[P]] = xv_ref[r0:r0 + nr].astype(jnp.bfloat16)

        def snd(P, i, start, n, msk):
            r = pltpu.make_async_remote_copy(
                src_ref=refs[P].at[pl.ds(start, n)],
                dst_ref=refs[P].at[pl.ds(start, n)],
                send_sem=send_sems.at[P, i],
                recv_sem=recv_sems.at[P, i],
                device_id=(jnp.bitwise_xor(my, msk),),
                device_id_type=pl.DeviceIdType.MESH,
            )
            r.start()
            return r

        def rcv(P, i, start, n):
            r = pltpu.make_async_remote_copy(
                src_ref=refs[P].at[pl.ds(start, n)],
                dst_ref=refs[P].at[pl.ds(start, n)],
                send_sem=send_sems.at[P, i],
                recv_sem=recv_sems.at[P, i],
                device_id=(my,),
                device_id_type=pl.DeviceIdType.MESH,
            )
            r.wait_recv()

        X = jnp.bitwise_xor
        sends = []
        for P, ((m0, m1, m2), _) in enumerate(PIPES):
            sends.append(snd(P, 0, ts[P], 1, m0))
            sends.append(snd(P, 1, ts[P], 1, m1))
            sends.append(snd(P, 3, ts[P], 1, m2))
        for P, ((m0, m1, m2), _) in enumerate(PIPES):
            rcv(P, 0, X(ts[P], 1), 1)
            sends.append(snd(P, 2, X(ts[P], 1), 1, m1))
            sends.append(snd(P, 4, X(ts[P], 1), 1, m2))
        for P, ((m0, m1, m2), _) in enumerate(PIPES):
            rcv(P, 1, X(ts[P], 2), 1)
            rcv(P, 2, X(ts[P], 3), 1)
            a2 = X(jnp.bitwise_and(ts[P], -2), 2)
            sends.append(snd(P, 5, a2, 2, m2))

        wcp.wait()
        w = wv_ref[...].astype(jnp.bfloat16)
        for P, (_, perm) in enumerate(PIPES):
            rcv(P, 3, X(ts[P], 4), 1)
            rcv(P, 4, X(ts[P], 5), 1)
            rcv(P, 5, X(jnp.bitwise_and(ts[P], -2), 6), 2)
            r0, nr = SPLITS[P]
            for s in range(N_DEV):
                y = jnp.dot(refs[P][perm[s]], w,
                            preferred_element_type=jnp.float32)
                out_ref[pl.ds(s * m_per + r0, nr), :] = (
                    y * jax.nn.sigmoid(y))

        for s_ in sends:
            s_.wait_send()

        @functools.partial(pl.run_scoped, sem2=pltpu.SemaphoreType.REGULAR)
        def _(sem2):
            for msk in (1, 3, 4):
                pl.semaphore_signal(
                    sem2, inc=1,
                    device_id=(jnp.bitwise_xor(my, msk),),
                    device_id_type=pl.DeviceIdType.MESH,
                )
            pl.semaphore_wait(sem2, 3)

    return pl.pallas_call(
        body,
        out_shape=jax.ShapeDtypeStruct((N_DEV * m_per, n_per), jnp.float32),
        in_specs=[
            pl.BlockSpec(memory_space=pl.ANY),
            pl.BlockSpec(memory_space=pl.ANY),
        ],
        out_specs=pl.BlockSpec(memory_space=pltpu.VMEM),
        scratch_shapes=[
            pltpu.VMEM((N_DEV, SPLITS[0][1], k), jnp.bfloat16),
            pltpu.VMEM((N_DEV, SPLITS[1][1], k), jnp.bfloat16),
            pltpu.VMEM((N_DEV, SPLITS[2][1], k), jnp.bfloat16),
            pltpu.VMEM((m_per, k), jnp.float32),
            pltpu.VMEM((k, n_per), jnp.float32),
            pltpu.SemaphoreType.DMA((3, 6)),
            pltpu.SemaphoreType.DMA((3, 6)),
            pltpu.SemaphoreType.DMA((2,)),
        ],
        compiler_params=pltpu.CompilerParams(collective_id=0),
    )(x, w_mat)


# device time: 42281 ns/iter; 1.0446x vs baseline; 1.0446x over previous
import functools

import jax
import jax.numpy as jnp
from jax import lax
from jax.experimental import pallas as pl
from jax.experimental.pallas import tpu as pltpu

N_DEV = 8

PIPES = (
    ((4, 3, 1), [0, 4, 6, 2, 1, 5, 7, 3]),
    ((3, 1, 4), [0, 2, 3, 1, 4, 6, 7, 5]),
    ((1, 4, 3), [0, 1, 5, 4, 2, 3, 7, 6]),
)
SPLITS = ((0, 96), (96, 80), (176, 80))


def kernel(x, w_mat):
    m_per, k = x.shape
    _, n_per = w_mat.shape

    def body(x_ref, w_ref, out_ref, xg0_ref, xg1_ref, xg2_ref,
             send_sems, recv_sems):
        my = lax.axis_index("i")
        b0 = jnp.bitwise_and(my, 1)
        b1 = jnp.bitwise_and(jnp.right_shift(my, 1), 1)
        b2 = jnp.bitwise_and(jnp.right_shift(my, 2), 1)
        b01 = jnp.bitwise_xor(b0, b1)
        refs = [xg0_ref, xg1_ref, xg2_ref]
        ts = [b2 + 2 * b1 + 4 * b01, b1 + 2 * b01 + 4 * b2,
              b01 + 2 * b2 + 4 * b1]

        barrier = pltpu.get_barrier_semaphore()
        for msk in (1, 3, 4):
            pl.semaphore_signal(
                barrier, inc=1,
                device_id=(jnp.bitwise_xor(my, msk),),
                device_id_type=pl.DeviceIdType.MESH,
            )
        pl.semaphore_wait(barrier, 3)

        xbf = x_ref[...].astype(jnp.bfloat16)
        for P in range(3):
            r0, nr = SPLITS[P]
            refs[P][ts[P]] = xbf[r0:r0 + nr]

        def snd(P, i, start, n, msk):
            r = pltpu.make_async_remote_copy(
                src_ref=refs[P].at[pl.ds(start, n)],
                dst_ref=refs[P].at[pl.ds(start, n)],
                send_sem=send_sems.at[P, i],
                recv_sem=recv_sems.at[P, i],
                device_id=(jnp.bitwise_xor(my, msk),),
                device_id_type=pl.DeviceIdType.MESH,
            )
            r.start()
            return r

        def rcv(P, i, start, n):
            r = pltpu.make_async_remote_copy(
                src_ref=refs[P].at[pl.ds(start, n)],
                dst_ref=refs[P].at[pl.ds(start, n)],
                send_sem=send_sems.at[P, i],
                recv_sem=recv_sems.at[P, i],
                device_id=(my,),
                device_id_type=pl.DeviceIdType.MESH,
            )
            r.wait_recv()

        X = jnp.bitwise_xor
        sends = []
        for P, ((m0, m1, m2), _) in enumerate(PIPES):
            sends.append(snd(P, 0, ts[P], 1, m0))
            sends.append(snd(P, 1, ts[P], 1, m1))
            sends.append(snd(P, 3, ts[P], 1, m2))
        for P, ((m0, m1, m2), _) in enumerate(PIPES):
            rcv(P, 0, X(ts[P], 1), 1)
            sends.append(snd(P, 2, X(ts[P], 1), 1, m1))
            sends.append(snd(P, 4, X(ts[P], 1), 1, m2))
        for P, ((m0, m1, m2), _) in enumerate(PIPES):
            rcv(P, 1, X(ts[P], 2), 1)
            rcv(P, 2, X(ts[P], 3), 1)
            a2 = X(jnp.bitwise_and(ts[P], -2), 2)
            sends.append(snd(P, 5, a2, 2, m2))

        def inv(P, v):
            t0 = jnp.bitwise_and(v, 1)
            t1 = jnp.bitwise_and(jnp.right_shift(v, 1), 1)
            t2 = jnp.bitwise_and(jnp.right_shift(v, 2), 1)
            if P == 0:
                return X(t2, t1) + 2 * t1 + 4 * t0
            if P == 1:
                return X(t1, t0) + 2 * t0 + 4 * t2
            return X(t0, t2) + 2 * t2 + 4 * t1

        def stripe(P, v):
            r0, nr = SPLITS[P]
            row = inv(P, v) * m_per + r0
            y = jnp.dot(refs[P][v], w, preferred_element_type=jnp.float32)
            out_ref[pl.ds(row, nr), :] = y * jax.nn.sigmoid(y)

        w = w_ref[...].astype(jnp.bfloat16)
        for P in range(3):
            for v in (ts[P], X(ts[P], 1), X(ts[P], 2), X(ts[P], 3)):
                stripe(P, v)

        for P in range(3):
            rcv(P, 3, X(ts[P], 4), 1)
            stripe(P, X(ts[P], 4))
            rcv(P, 4, X(ts[P], 5), 1)
            stripe(P, X(ts[P], 5))
            a6 = X(jnp.bitwise_and(ts[P], -2), 6)
            rcv(P, 5, a6, 2)
            stripe(P, a6)
            stripe(P, X(a6, 1))

        for s_ in sends:
            s_.wait_send()

        @functools.partial(pl.run_scoped, sem2=pltpu.SemaphoreType.REGULAR)
        def _(sem2):
            for msk in (1, 3, 4):
                pl.semaphore_signal(
                    sem2, inc=1,
                    device_id=(jnp.bitwise_xor(my, msk),),
                    device_id_type=pl.DeviceIdType.MESH,
                )
            pl.semaphore_wait(sem2, 3)

    return pl.pallas_call(
        body,
        out_shape=jax.ShapeDtypeStruct((N_DEV * m_per, n_per), jnp.float32),
        in_specs=[
            pl.BlockSpec(memory_space=pltpu.VMEM),
            pl.BlockSpec(memory_space=pltpu.VMEM),
        ],
        out_specs=pl.BlockSpec(memory_space=pltpu.VMEM),
        scratch_shapes=[
            pltpu.VMEM((N_DEV, SPLITS[0][1], k), jnp.bfloat16),
            pltpu.VMEM((N_DEV, SPLITS[1][1], k), jnp.bfloat16),
            pltpu.VMEM((N_DEV, SPLITS[2][1], k), jnp.bfloat16),
            pltpu.SemaphoreType.DMA((3, 6)),
            pltpu.SemaphoreType.DMA((3, 6)),
        ],
        compiler_params=pltpu.CompilerParams(collective_id=0),
    )(x, w_mat)
